# speedup vs baseline: 6.6788x; 6.6788x over previous
import functools

import jax
import jax.numpy as jnp
from jax import lax
from jax.experimental import pallas as pl
from jax.experimental.pallas import tpu as pltpu
from jax.experimental.pallas import tpu_sc as plsc

H = 128
NUM_POSI = 512
NUM_AGE = 120

_NC = 2
_NS = 16
_NW = _NC * _NS

_CHUNK = 128


def _sc_word_gather(table, idx_flat):
    n = idx_flat.shape[0]
    idx2d = idx_flat.reshape(1, n)
    mesh = plsc.VectorSubcoreMesh(core_axis_name="c", subcore_axis_name="s")

    @functools.partial(
        pl.kernel,
        out_type=jax.ShapeDtypeStruct((n, H), jnp.float32),
        mesh=mesh,
    )
    def gather_kernel(table_hbm, idx_hbm, out_hbm):
        def body(idx_vmem, out_vmem):
            pltpu.sync_copy(table_hbm.at[idx_vmem.at[0]], out_vmem)

        pltpu.emit_pipeline(
            body,
            grid=(n // _CHUNK,),
            in_specs=[pl.BlockSpec((1, _CHUNK), lambda i: (0, i))],
            out_specs=[pl.BlockSpec((_CHUNK, H), lambda i: (i, 0))],
            core_axis_name=("c", "s"),
            dimension_semantics=(pltpu.PARALLEL,),
        )(idx_hbm, out_hbm)

    return gather_kernel(table, idx2d)


_ROWS = 512


def _tc_body(code_ref, w_ref, posi_ref, age_ref, seg_ref, g_ref, b_ref, o_ref):
    code = code_ref[0, 0, :]
    p = code & (NUM_POSI - 1)
    a = (code >> 9) & 127
    s = (code >> 16) & 1

    kp = lax.broadcasted_iota(jnp.int32, (_ROWS, NUM_POSI), 1)
    oh_p = jnp.where(kp == p[:, None], 1.0, 0.0).astype(jnp.bfloat16)
    ka = lax.broadcasted_iota(jnp.int32, (_ROWS, 128), 1)
    oh_a = jnp.where(ka == a[:, None], 1.0, 0.0).astype(jnp.bfloat16)

    e_p = jnp.dot(oh_p, posi_ref[...], preferred_element_type=jnp.float32)
    e_a = jnp.dot(oh_a, age_ref[...], preferred_element_type=jnp.float32)

    seg0 = seg_ref[0:1, :]
    seg_diff = seg_ref[1:2, :] - seg0
    e_s = seg0 + s[:, None].astype(jnp.float32) * seg_diff

    e = w_ref[...] + e_p + e_a + e_s
    mu = jnp.mean(e, axis=-1, keepdims=True)
    var = jnp.mean((e - mu) ** 2, axis=-1, keepdims=True)
    o_ref[...] = (e - mu) * lax.rsqrt(var + 1e-12) * g_ref[...] + b_ref[...]


def _tc_fuse(codes, wordemb, posi_bf16, age_pad_bf16, w_seg, ln_gamma, ln_beta):
    n = wordemb.shape[0]
    nb = n // _ROWS
    codes3d = codes.reshape(nb, 1, _ROWS)
    return pl.pallas_call(
        _tc_body,
        grid=(nb,),
        in_specs=[
            pl.BlockSpec((1, 1, _ROWS), lambda i: (i, 0, 0)),
            pl.BlockSpec((_ROWS, H), lambda i: (i, 0)),
            pl.BlockSpec((NUM_POSI, H), lambda i: (0, 0)),
            pl.BlockSpec((128, H), lambda i: (0, 0)),
            pl.BlockSpec((2, H), lambda i: (0, 0)),
            pl.BlockSpec((1, H), lambda i: (0, 0)),
            pl.BlockSpec((1, H), lambda i: (0, 0)),
        ],
        out_specs=pl.BlockSpec((_ROWS, H), lambda i: (i, 0)),
        out_shape=jax.ShapeDtypeStruct((n, H), jnp.float32),
    )(codes3d, wordemb, posi_bf16, age_pad_bf16, w_seg,
      ln_gamma.reshape(1, H), ln_beta.reshape(1, H))


def kernel(word_ids, age_ids, seg_ids, posi_ids, W_word, W_seg, W_age, W_posi,
           ln_gamma, ln_beta):
    b, l = word_ids.shape
    n = b * l

    wordemb = _sc_word_gather(W_word, word_ids.reshape(n).astype(jnp.int32))

    codes = (posi_ids.astype(jnp.int32)
             | (age_ids.astype(jnp.int32) << 9)
             | (seg_ids.astype(jnp.int32) << 16)).reshape(n)

    posi_bf16 = W_posi.astype(jnp.bfloat16)
    age_pad_bf16 = jnp.zeros((128, H), jnp.bfloat16).at[:NUM_AGE].set(
        W_age.astype(jnp.bfloat16))

    out = _tc_fuse(codes, wordemb, posi_bf16, age_pad_bf16, W_seg,
                   ln_gamma, ln_beta)
    return out.reshape(b, l, H)
